# Initial kernel scaffold; baseline (speedup 1.0000x reference)
import jax
import jax.numpy as jnp
from jax.experimental import pallas as pl


def kernel(nf1, nf2, nf3, nf4, disjoint, nf3_neg0, nf3_neg1, class_embeds, relation_embeds):
    raise NotImplementedError("write your pallas kernel here")


# speedup vs baseline: 1.2564x; 1.2564x over previous
import functools

import jax
import jax.numpy as jnp
from jax import lax
from jax.experimental import pallas as pl
from jax.experimental.pallas import tpu as pltpu
from jax.experimental.pallas import tpu_sc as plsc

DIM = 128
BATCH = 512
MARGIN = 0.0
NEG_DIST = 2.0

NC = 2
NS = 16
NW = NC * NS

CLS_B = 15 * BATCH
REL_B = 4 * BATCH

CLS_PER_W = CLS_B // NW
REL_PER_W = REL_B // NW
CLS_CHUNK = CLS_PER_W // 2


def _gather_body(ce_hbm, re_hbm, ci_hbm, ri_hbm, cls_out, rel_out,
                 ci_v, crows_v, ri_v, rrows_v, sem):
    wid = lax.axis_index("s") * NC + lax.axis_index("c")
    cbase = wid * CLS_PER_W
    for j in range(2):
        b = cbase + j * CLS_CHUNK
        pltpu.sync_copy(ci_hbm.at[pl.ds(b, CLS_CHUNK)], ci_v)
        pltpu.async_copy(ce_hbm.at[ci_v], crows_v, sem).wait()
        pltpu.sync_copy(crows_v, cls_out.at[pl.ds(b, CLS_CHUNK)])
    rbase = wid * REL_PER_W
    pltpu.sync_copy(ri_hbm.at[pl.ds(rbase, REL_PER_W)], ri_v)
    pltpu.async_copy(re_hbm.at[ri_v], rrows_v, sem).wait()
    pltpu.sync_copy(rrows_v, rel_out.at[pl.ds(rbase, REL_PER_W)])


@functools.partial(jax.jit, static_argnums=())
def _gather_sc(class_embeds, relation_embeds, cls_idx, rel_idx):
    mesh = plsc.VectorSubcoreMesh(core_axis_name="c", subcore_axis_name="s",
                                  num_cores=NC, num_subcores=NS)
    f = pl.kernel(
        _gather_body,
        out_type=[
            jax.ShapeDtypeStruct((CLS_B, 2 * DIM), jnp.float32),
            jax.ShapeDtypeStruct((REL_B, DIM), jnp.float32),
        ],
        mesh=mesh,
        scratch_types=[
            pltpu.VMEM((CLS_CHUNK,), jnp.int32),
            pltpu.VMEM((CLS_CHUNK, 2 * DIM), jnp.float32),
            pltpu.VMEM((REL_PER_W,), jnp.int32),
            pltpu.VMEM((REL_PER_W, DIM), jnp.float32),
            pltpu.SemaphoreType.DMA,
        ],
    )
    return f(class_embeds, relation_embeds, cls_idx, rel_idx)


def _ssq_relu(x):
    return jnp.sum(jnp.square(jax.nn.relu(x)), axis=1)


def _loss_body(cls_ref, rel_ref, out_ref):
    e = cls_ref[...]
    c = e[:, :DIM]
    o = jnp.abs(e[:, DIM:])
    rel = rel_ref[...]
    S = BATCH

    def cs(k, n=1):
        return c[k * S:(k + n) * S], o[k * S:(k + n) * S]

    cc, co = cs(0)
    dc, do = cs(1)
    loss1 = jnp.mean(_ssq_relu(jnp.abs(cc - dc) + co - do))

    cc, co = cs(2)
    dc, do = cs(3)
    ec, eo = cs(4)
    lower = jnp.maximum(cc - co, dc - do)
    upper = jnp.minimum(cc + co, dc + do)
    ic = (lower + upper) * 0.5
    io = jnp.abs(upper - lower) * 0.5
    a = jnp.sqrt(_ssq_relu(jnp.abs(ic - ec) + io - eo))
    b = jnp.sqrt(_ssq_relu(lower - upper))
    loss2 = (jnp.mean(jnp.square(a)) + jnp.mean(jnp.square(b))
             + 2.0 * jnp.mean(a) * jnp.mean(b))

    cc, co = cs(5)
    dc, do = cs(6)
    r = rel[0 * S:1 * S]
    loss3 = jnp.mean(_ssq_relu(jnp.abs(cc + r - dc) + co - do))

    cc, co = cs(7)
    dc, do = cs(8)
    r = rel[1 * S:2 * S]
    loss4 = jnp.mean(_ssq_relu(jnp.abs(cc - r - dc) + co - do))

    cc, co = cs(9)
    dc, do = cs(10)
    loss_dis = jnp.mean(_ssq_relu(-jnp.abs(cc - dc) + co + do))

    cc, co = cs(11, 2)
    dc, do = cs(13, 2)
    r = rel[2 * S:4 * S]
    nneg = jnp.sqrt(_ssq_relu(jnp.abs(cc + r - dc) - co - do))
    neg_loss = jnp.mean(jnp.square(NEG_DIST - nneg))

    total = loss1 + loss2 + loss_dis + loss3 + loss4 + neg_loss
    out_ref[...] = jnp.reshape(total, (1, 1))


def _loss_tc(cls_rows, rel_rows):
    return pl.pallas_call(
        _loss_body,
        out_shape=jax.ShapeDtypeStruct((1, 1), jnp.float32),
    )(cls_rows, rel_rows)


def kernel(nf1, nf2, nf3, nf4, disjoint, nf3_neg0, nf3_neg1,
           class_embeds, relation_embeds):
    kk = jax.random.split(jax.random.key(42), 6)
    i1 = jax.random.randint(kk[0], (BATCH,), 0, nf1.shape[0])
    i2 = jax.random.randint(kk[1], (BATCH,), 0, nf2.shape[0])
    i3 = jax.random.randint(kk[2], (BATCH,), 0, nf3.shape[0])
    i4 = jax.random.randint(kk[3], (BATCH,), 0, nf4.shape[0])
    i5 = jax.random.randint(kk[4], (BATCH,), 0, disjoint.shape[0])
    i6 = jax.random.randint(kk[5], (BATCH,), 0, nf3_neg0.shape[0])
    d1 = nf1[i1]
    d2 = nf2[i2]
    d3 = nf3[i3]
    d4 = nf4[i4]
    d5 = disjoint[i5]
    n0 = nf3_neg0[i6]
    n1 = nf3_neg1[i6]
    cls_idx = jnp.concatenate([
        d1[:, 0], d1[:, 1],
        d2[:, 0], d2[:, 1], d2[:, 2],
        d3[:, 0], d3[:, 2],
        d4[:, 1], d4[:, 2],
        d5[:, 0], d5[:, 1],
        n0[:, 0], n1[:, 0],
        n0[:, 2], n1[:, 2],
    ]).astype(jnp.int32)
    rel_idx = jnp.concatenate([
        d3[:, 1], d4[:, 0], n0[:, 1], n1[:, 1],
    ]).astype(jnp.int32)
    cls_rows, rel_rows = _gather_sc(class_embeds, relation_embeds,
                                    cls_idx, rel_idx)
    return _loss_tc(cls_rows, rel_rows)[0, 0]
